# baseline (speedup 1.0000x reference)
import functools
import math

import jax
import jax.numpy as jnp
from jax import lax
from jax.experimental import pallas as pl
from jax.experimental.pallas import tpu as pltpu
from jax.experimental.pallas import tpu_sc as plsc

N = 20000
NQ = 16
MS = (1, 3, 5, 7)
KS = tuple(m * NQ for m in MS)
LOFF = (0, 256, 512, 768)
OUT_D = 1024
BR = 16
NBLK = N // BR
NW = 32
RSCALE = tuple(math.sqrt(1.0 / math.sqrt(2 * l + 1)) for l in range(4))
PBS = (16, 8, 4, 4)

_mesh = plsc.VectorSubcoreMesh(core_axis_name="c", subcore_axis_name="s")


@functools.partial(
    pl.kernel,
    mesh=_mesh,
    compiler_params=pltpu.CompilerParams(needs_layout_passes=False),
    out_type=jax.ShapeDtypeStruct((N, OUT_D), jnp.float32),
    scratch_types=[
        pltpu.VMEM((BR, KS[0]), jnp.float32),
        pltpu.VMEM((BR, KS[1]), jnp.float32),
        pltpu.VMEM((BR, KS[2]), jnp.float32),
        pltpu.VMEM((BR, KS[3]), jnp.float32),
        pltpu.VMEM((KS[0], NQ), jnp.float32),
        pltpu.VMEM((KS[1], NQ), jnp.float32),
        pltpu.VMEM((KS[2], NQ), jnp.float32),
        pltpu.VMEM((KS[3], NQ), jnp.float32),
        pltpu.VMEM((BR, OUT_D), jnp.float32),
        pltpu.SemaphoreType.DMA,
    ],
)
def _ps_kernel(v0, v1, v2, v3, out, b0, b1, b2, b3, t0, t1, t2, t3, ob, sem):
    cid = lax.axis_index("c")
    sid = lax.axis_index("s")
    wid = sid * 2 + cid
    nblk_w = 39 + (wid < (NBLK - NW * (NBLK // NW))).astype(jnp.int32)

    iota = lax.iota(jnp.int32, NQ)
    row_off = iota * OUT_D

    vs = (v0, v1, v2, v3)
    ins = (b0, b1, b2, b3)
    ts = (t0, t1, t2, t3)

    def block_body(i, carry):
        r0 = (wid + i * NW) * BR
        copies = [
            pltpu.async_copy(vs[l].at[pl.ds(r0, BR)], ins[l], sem)
            for l in range(4)
        ]
        for c in copies:
            c.wait()

        for l in range(4):
            t, b = ts[l], ins[l]
            scale = RSCALE[l]

            @plsc.parallel_loop(0, KS[l], unroll=8)
            def tbody(j, b=b, t=t, scale=scale):
                col = plsc.load_gather(b, [iota, jnp.full((NQ,), j, jnp.int32)])
                if scale != 1.0:
                    col = col * scale
                t[j] = col

        for l in range(4):
            t = ts[l]
            M = MS[l]
            PB = PBS[l]
            for p0 in range(0, NQ, PB):
                bcols = [[t[m * NQ + p0 + j] for j in range(PB)] for m in range(M)]

                @plsc.parallel_loop(0, NQ, unroll=2)
                def qbody(q, l=l, M=M, PB=PB, p0=p0, bcols=bcols, t=t):
                    acols = [t[m * NQ + q] for m in range(M)]
                    base = LOFF[l] + q * NQ + p0
                    for j in range(PB):
                        acc = acols[0] * bcols[0][j]
                        for m in range(1, M):
                            acc = acc + acols[m] * bcols[m][j]
                        colv = jnp.full((NQ,), base + j, jnp.int32)
                        plsc.store_scatter(ob, [iota, colv], acc)

        pltpu.sync_copy(ob, out.at[pl.ds(r0, BR)])
        return carry

    lax.fori_loop(0, nblk_w, block_body, 0)


def kernel(values_l0, values_l1, values_l2, values_l3):
    return _ps_kernel(
        values_l0.reshape(N, KS[0]),
        values_l1.reshape(N, KS[1]),
        values_l2.reshape(N, KS[2]),
        values_l3.reshape(N, KS[3]),
    )


# speedup vs baseline: 1.6530x; 1.6530x over previous
import functools
import math

import jax
import jax.numpy as jnp
from jax import lax
from jax.experimental import pallas as pl
from jax.experimental.pallas import tpu as pltpu
from jax.experimental.pallas import tpu_sc as plsc

N = 20000
NQ = 16
MS = (1, 3, 5, 7)
KS = tuple(m * NQ for m in MS)
LOFF = (0, 256, 512, 768)
OUT_D = 1024
BR = 16
NBLK = N // BR
NW = 32
CG = tuple(1.0 / math.sqrt(2 * l + 1) for l in range(4))

_mesh = plsc.VectorSubcoreMesh(core_axis_name="c", subcore_axis_name="s")


@functools.partial(
    pl.kernel,
    mesh=_mesh,
    compiler_params=pltpu.CompilerParams(needs_layout_passes=False),
    out_type=jax.ShapeDtypeStruct((N, OUT_D), jnp.float32),
    scratch_types=[
        pltpu.VMEM((BR, KS[0]), jnp.float32),
        pltpu.VMEM((BR, KS[1]), jnp.float32),
        pltpu.VMEM((BR, KS[2]), jnp.float32),
        pltpu.VMEM((BR, KS[3]), jnp.float32),
        pltpu.VMEM((BR, OUT_D), jnp.float32),
        pltpu.SemaphoreType.DMA,
    ],
)
def _ps_kernel(v0, v1, v2, v3, out, b0, b1, b2, b3, ob, sem):
    cid = lax.axis_index("c")
    sid = lax.axis_index("s")
    wid = sid * 2 + cid
    nblk_w = 39 + (wid < (NBLK - NW * (NBLK // NW))).astype(jnp.int32)

    vs = (v0, v1, v2, v3)
    ins = (b0, b1, b2, b3)
    idxq = [jnp.full((NQ,), q, jnp.int32) for q in range(NQ)]

    def block_body(i, carry):
        r0 = (wid + i * NW) * BR
        copies = [
            pltpu.async_copy(vs[l].at[pl.ds(r0, BR)], ins[l], sem)
            for l in range(4)
        ]
        for c in copies:
            c.wait()

        @plsc.parallel_loop(0, BR, unroll=1)
        def rbody(r):
            for l in range(4):
                M = MS[l]
                raw = [ins[l][r, pl.ds(mm * NQ, NQ)] for mm in range(M)]
                if CG[l] != 1.0:
                    sc = [v * CG[l] for v in raw]
                else:
                    sc = raw
                for q in range(NQ):
                    b = jnp.take_along_axis(sc[0], idxq[q], axis=0)
                    acc = b * raw[0]
                    for mm in range(1, M):
                        b = jnp.take_along_axis(sc[mm], idxq[q], axis=0)
                        acc = acc + b * raw[mm]
                    ob[r, pl.ds(LOFF[l] + q * NQ, NQ)] = acc

        pltpu.sync_copy(ob, out.at[pl.ds(r0, BR)])
        return carry

    lax.fori_loop(0, nblk_w, block_body, 0)


def kernel(values_l0, values_l1, values_l2, values_l3):
    return _ps_kernel(
        values_l0.reshape(N, KS[0]),
        values_l1.reshape(N, KS[1]),
        values_l2.reshape(N, KS[2]),
        values_l3.reshape(N, KS[3]),
    )


# speedup vs baseline: 2.0052x; 1.2131x over previous
import functools
import math

import jax
import jax.numpy as jnp
from jax import lax
from jax.experimental import pallas as pl
from jax.experimental.pallas import tpu as pltpu
from jax.experimental.pallas import tpu_sc as plsc

N = 20000
NQ = 16
MS = (1, 3, 5, 7)
KS = tuple(m * NQ for m in MS)
LOFF = (0, 256, 512, 768)
OUT_D = 1024
BR = 16
NBLK = N // BR
NW = 32
CG = tuple(1.0 / math.sqrt(2 * l + 1) for l in range(4))

_mesh = plsc.VectorSubcoreMesh(core_axis_name="c", subcore_axis_name="s")

_IN_SCRATCH = [
    pltpu.VMEM((BR, KS[l]), jnp.float32) for l in range(4)
]


@functools.partial(
    pl.kernel,
    mesh=_mesh,
    compiler_params=pltpu.CompilerParams(needs_layout_passes=False),
    out_type=jax.ShapeDtypeStruct((N, OUT_D), jnp.float32),
    scratch_types=[
        *_IN_SCRATCH,
        *_IN_SCRATCH,
        pltpu.VMEM((BR, OUT_D), jnp.float32),
        pltpu.VMEM((BR, OUT_D), jnp.float32),
        pltpu.SemaphoreType.DMA,
        pltpu.SemaphoreType.DMA,
        pltpu.SemaphoreType.DMA,
        pltpu.SemaphoreType.DMA,
    ],
)
def _ps_kernel(v0, v1, v2, v3, out,
               a0, a1, a2, a3, c0, c1, c2, c3, ob0, ob1,
               si0, si1, so0, so1):
    cid = lax.axis_index("c")
    sid = lax.axis_index("s")
    wid = sid * 2 + cid
    nblk_w = 39 + (wid < (NBLK - NW * (NBLK // NW))).astype(jnp.int32)

    vs = (v0, v1, v2, v3)
    ins = ((a0, a1, a2, a3), (c0, c1, c2, c3))
    obs = (ob0, ob1)
    sin = (si0, si1)
    sout = (so0, so1)
    idxq = [jnp.full((NQ,), q, jnp.int32) for q in range(NQ)]

    def blk_r0(b):
        return (wid + b * NW) * BR

    def issue_in(b, s):
        for l in range(4):
            pltpu.async_copy(vs[l].at[pl.ds(blk_r0(b), BR)], ins[s][l], sin[s])

    def wait_in(b, s):
        for l in range(4):
            pltpu.make_async_copy(
                vs[l].at[pl.ds(blk_r0(b), BR)], ins[s][l], sin[s]
            ).wait()

    def compute(s):
        bufs = ins[s]
        ob = obs[s]

        @plsc.parallel_loop(0, BR, unroll=1)
        def rbody(r):
            for l in range(4):
                M = MS[l]
                raw = [bufs[l][r, pl.ds(mm * NQ, NQ)] for mm in range(M)]
                if CG[l] != 1.0:
                    scv = [v * CG[l] for v in raw]
                else:
                    scv = raw
                for q in range(NQ):
                    bq = jnp.take_along_axis(scv[0], idxq[q], axis=0)
                    acc = bq * raw[0]
                    for mm in range(1, M):
                        bq = jnp.take_along_axis(scv[mm], idxq[q], axis=0)
                        acc = acc + bq * raw[mm]
                    ob[r, pl.ds(LOFF[l] + q * NQ, NQ)] = acc

    def issue_out(b, s):
        pltpu.async_copy(obs[s], out.at[pl.ds(blk_r0(b), BR)], sout[s])

    def wait_out(b, s):
        pltpu.make_async_copy(
            obs[s], out.at[pl.ds(blk_r0(b), BR)], sout[s]
        ).wait()

    issue_in(0, 0)

    def pair_body(i2, carry):
        e = 2 * i2
        o = e + 1

        @pl.when(o < nblk_w)
        def _():
            issue_in(o, 1)

        wait_in(e, 0)

        @pl.when(i2 > 0)
        def _():
            wait_out(e - 2, 0)

        compute(0)
        issue_out(e, 0)

        @pl.when(o < nblk_w)
        def _():
            @pl.when(o + 1 < nblk_w)
            def _():
                issue_in(o + 1, 0)

            wait_in(o, 1)

            @pl.when(i2 > 0)
            def _():
                wait_out(o - 2, 1)

            compute(1)
            issue_out(o, 1)

        return carry

    lax.fori_loop(0, 20, pair_body, 0)
    wait_out(0, 0)
    wait_out(0, 1)


def kernel(values_l0, values_l1, values_l2, values_l3):
    return _ps_kernel(
        values_l0.reshape(N, KS[0]),
        values_l1.reshape(N, KS[1]),
        values_l2.reshape(N, KS[2]),
        values_l3.reshape(N, KS[3]),
    )


# speedup vs baseline: 3.1655x; 1.5787x over previous
import functools
import math

import jax
import jax.numpy as jnp
from jax import lax
from jax.experimental import pallas as pl
from jax.experimental.pallas import tpu as pltpu
from jax.experimental.pallas import tpu_sc as plsc

N = 20000
NQ = 16
MS = (1, 3, 5, 7)
KS = tuple(m * NQ for m in MS)
LOFF = (0, 256, 512, 768)
OUT_D = 1024
BR = 16
NBLK = N // BR
NW = 32
CG = tuple(1.0 / math.sqrt(2 * l + 1) for l in range(4))

_mesh = plsc.VectorSubcoreMesh(core_axis_name="c", subcore_axis_name="s")

_IN_SCRATCH = [
    pltpu.VMEM((BR, KS[l]), jnp.float32) for l in range(4)
]


@functools.partial(
    pl.kernel,
    mesh=_mesh,
    compiler_params=pltpu.CompilerParams(needs_layout_passes=False),
    out_type=jax.ShapeDtypeStruct((N, OUT_D), jnp.float32),
    scratch_types=[
        *_IN_SCRATCH,
        *_IN_SCRATCH,
        pltpu.VMEM((BR, OUT_D), jnp.float32),
        pltpu.VMEM((BR, OUT_D), jnp.float32),
        pltpu.SemaphoreType.DMA,
        pltpu.SemaphoreType.DMA,
        pltpu.SemaphoreType.DMA,
        pltpu.SemaphoreType.DMA,
    ],
)
def _ps_kernel(v0, v1, v2, v3, out,
               a0, a1, a2, a3, c0, c1, c2, c3, ob0, ob1,
               si0, si1, so0, so1):
    cid = lax.axis_index("c")
    sid = lax.axis_index("s")
    wid = sid * 2 + cid
    nblk_w = 39 + (wid < (NBLK - NW * (NBLK // NW))).astype(jnp.int32)

    vs = (v0, v1, v2, v3)
    ins = ((a0, a1, a2, a3), (c0, c1, c2, c3))
    obs = (ob0, ob1)
    sin = (si0, si1)
    sout = (so0, so1)

    def blk_r0(b):
        return (wid + b * NW) * BR

    def issue_in(b, s):
        for l in range(4):
            pltpu.async_copy(vs[l].at[pl.ds(blk_r0(b), BR)], ins[s][l], sin[s])

    def wait_in(b, s):
        for l in range(4):
            pltpu.make_async_copy(
                vs[l].at[pl.ds(blk_r0(b), BR)], ins[s][l], sin[s]
            ).wait()

    idxq = [jnp.full((NQ,), q, jnp.int32) for q in range(NQ)]

    def compute(s):
        bufs = ins[s]
        ob = obs[s]

        @plsc.parallel_loop(0, BR, unroll=2)
        def rbody(r):
            for l in range(4):
                M = MS[l]
                raw = [bufs[l][r, pl.ds(mm * NQ, NQ)] for mm in range(M)]
                if CG[l] != 1.0:
                    scv = [v * CG[l] for v in raw]
                else:
                    scv = raw
                for q in range(NQ):
                    bq = jnp.take_along_axis(scv[0], idxq[q], axis=0)
                    acc = bq * raw[0]
                    for mm in range(1, M):
                        bq = jnp.take_along_axis(scv[mm], idxq[q], axis=0)
                        acc = acc + bq * raw[mm]
                    ob[r, pl.ds(LOFF[l] + q * NQ, NQ)] = acc

    def issue_out(b, s):
        pltpu.async_copy(obs[s], out.at[pl.ds(blk_r0(b), BR)], sout[s])

    def wait_out(b, s):
        pltpu.make_async_copy(
            obs[s], out.at[pl.ds(blk_r0(b), BR)], sout[s]
        ).wait()

    issue_in(0, 0)

    def pair_body(i2, carry):
        e = 2 * i2
        o = e + 1

        @pl.when(o < nblk_w)
        def _():
            issue_in(o, 1)

        wait_in(e, 0)

        @pl.when(i2 > 0)
        def _():
            wait_out(e - 2, 0)

        compute(0)
        issue_out(e, 0)

        @pl.when(o < nblk_w)
        def _():
            @pl.when(o + 1 < nblk_w)
            def _():
                issue_in(o + 1, 0)

            wait_in(o, 1)

            @pl.when(i2 > 0)
            def _():
                wait_out(o - 2, 1)

            compute(1)
            issue_out(o, 1)

        return carry

    lax.fori_loop(0, 20, pair_body, 0)
    wait_out(0, 0)
    wait_out(0, 1)


def kernel(values_l0, values_l1, values_l2, values_l3):
    return _ps_kernel(
        values_l0.reshape(N, KS[0]),
        values_l1.reshape(N, KS[1]),
        values_l2.reshape(N, KS[2]),
        values_l3.reshape(N, KS[3]),
    )
